# initial kernel scaffold (unmeasured)
import jax
import jax.numpy as jnp
from jax import lax
from jax.experimental import pallas as pl
from jax.experimental.pallas import tpu as pltpu

N = 4096
D = 1024
C = 128
MAX_CHUNKS = N // C
TAIL = MAX_CHUNKS
NSEM = MAX_CHUNKS + 1


def _body(k_ref, x_ref, out_ref, send_sems, recv_sems):
    my_x = lax.axis_index("x")
    my_y = lax.axis_index("y")
    my_z = lax.axis_index("z")
    nbr = (my_x, 1 - my_y, my_z)

    k = k_ref[0]
    m = N - k
    base_local = jnp.where(my_y == 0, 0, k)
    base_remote = jnp.where(my_y == 0, 0, N - k)
    base_recv = jnp.where(my_y == 0, N - k, 0)

    barrier = pltpu.get_barrier_semaphore()
    pl.semaphore_signal(
        barrier, inc=1, device_id=nbr, device_id_type=pl.DeviceIdType.MESH
    )
    pl.semaphore_wait(barrier, 1)

    def send_rdma(src_off, dst_off, sem):
        return pltpu.make_async_remote_copy(
            src_ref=x_ref.at[pl.ds(src_off, C)],
            dst_ref=out_ref.at[pl.ds(dst_off, C)],
            send_sem=send_sems.at[sem],
            recv_sem=recv_sems.at[sem],
            device_id=nbr,
            device_id_type=pl.DeviceIdType.MESH,
        )

    def recv_rdma(dst_off, sem):
        return send_rdma(0, dst_off, sem)

    for j in range(MAX_CHUNKS):

        @pl.when((j + 1) * C <= k)
        def _(j=j):
            send_rdma(m + j * C, base_remote + j * C, j).start()

    @pl.when(k % C != 0)
    def _():
        send_rdma(N - C, base_remote + k - C, TAIL).start()

    for j in range(MAX_CHUNKS):

        @pl.when((j + 1) * C <= m)
        def _(j=j):
            out_ref[pl.ds(base_local + j * C, C), :] = x_ref[pl.ds(j * C, C), :]

    @pl.when(m % C != 0)
    def _():
        out_ref[pl.ds(base_local + m - C, C), :] = x_ref[pl.ds(m - C, C), :]

    for j in range(MAX_CHUNKS):

        @pl.when((j + 1) * C <= k)
        def _(j=j):
            recv_rdma(base_recv + j * C, j).wait_recv()

    @pl.when(k % C != 0)
    def _():
        recv_rdma(base_recv + k - C, TAIL).wait_recv()

    for j in range(MAX_CHUNKS):

        @pl.when((j + 1) * C <= k)
        def _(j=j):
            send_rdma(m + j * C, base_remote + j * C, j).wait_send()

    @pl.when(k % C != 0)
    def _():
        send_rdma(N - C, base_remote + k - C, TAIL).wait_send()


def kernel(x, dest):
    my_y = lax.axis_index("y")
    send_mask = (dest != my_y).astype(jnp.int32)
    k = jnp.sum(send_mask).astype(jnp.int32)
    perm = jnp.argsort(send_mask, stable=True)
    x_perm = x.astype(jnp.bfloat16)[perm]
    k_arr = jnp.reshape(k, (1,))

    return pl.pallas_call(
        _body,
        out_shape=jax.ShapeDtypeStruct((N, D), jnp.bfloat16),
        in_specs=[
            pl.BlockSpec(memory_space=pltpu.SMEM),
            pl.BlockSpec(memory_space=pltpu.VMEM),
        ],
        out_specs=pl.BlockSpec(memory_space=pltpu.VMEM),
        scratch_shapes=[
            pltpu.SemaphoreType.DMA((NSEM,)),
            pltpu.SemaphoreType.DMA((NSEM,)),
        ],
        compiler_params=pltpu.CompilerParams(collective_id=0),
    )(k_arr, x_perm)


# baseline (device time: 432260 ns/iter reference)
import jax
import jax.numpy as jnp
from jax import lax
from jax.experimental import pallas as pl
from jax.experimental.pallas import tpu as pltpu

N = 4096
D = 1024
C = 128
NCHUNK = N // C


def _body(k_ref, keep_ref, send_ref, out_ref, recv_buf, send_sems, recv_sems):
    my_x = lax.axis_index("x")
    my_y = lax.axis_index("y")
    my_z = lax.axis_index("z")
    nbr = (my_x, 1 - my_y, my_z)

    k = k_ref[0]
    base_remote = jnp.where(my_y == 0, 0, N - k)
    base_recv = jnp.where(my_y == 0, N - k, 0)

    barrier = pltpu.get_barrier_semaphore()
    pl.semaphore_signal(
        barrier, inc=1, device_id=nbr, device_id_type=pl.DeviceIdType.MESH
    )
    pl.semaphore_wait(barrier, 1)

    def chunk_rdma(j):
        return pltpu.make_async_remote_copy(
            src_ref=send_ref.at[pl.ds(j * C, C)],
            dst_ref=recv_buf.at[pl.ds(j * C, C)],
            send_sem=send_sems.at[j],
            recv_sem=recv_sems.at[j],
            device_id=nbr,
            device_id_type=pl.DeviceIdType.MESH,
        )

    def overlaps(j, base):
        return jnp.logical_and((j + 1) * C > base, j * C < base + k)

    for j in range(NCHUNK):

        @pl.when(overlaps(j, base_remote))
        def _(j=j):
            chunk_rdma(j).start()

    for j in range(NCHUNK):

        @pl.when(overlaps(j, base_recv))
        def _(j=j):
            chunk_rdma(j).wait_recv()

    row = lax.broadcasted_iota(jnp.int32, (N, 1), 0)
    in_recv = jnp.logical_and(row >= base_recv, row < base_recv + k)
    out_ref[...] = jnp.where(in_recv, recv_buf[...], keep_ref[...])

    for j in range(NCHUNK):

        @pl.when(overlaps(j, base_remote))
        def _(j=j):
            chunk_rdma(j).wait_send()


def kernel(x, dest):
    my_y = lax.axis_index("y")
    send_mask = (dest != my_y).astype(jnp.int32)
    k = jnp.sum(send_mask)
    base_local = jnp.where(my_y == 0, 0, k)
    base_remote = jnp.where(my_y == 0, 0, N - k)

    order_keep = jnp.argsort(send_mask, stable=True)
    order_send = jnp.argsort(1 - send_mask, stable=True)

    pos = jnp.arange(N, dtype=jnp.int32)
    src_keep = order_keep[jnp.clip(pos - base_local, 0, N - 1)]
    src_send = order_send[jnp.clip(pos - base_remote, 0, N - 1)]
    xb = x.astype(jnp.bfloat16)
    keep_buf = xb[src_keep]
    send_buf = xb[src_send]
    k_arr = jnp.reshape(k, (1,)).astype(jnp.int32)

    return pl.pallas_call(
        _body,
        out_shape=jax.ShapeDtypeStruct((N, D), jnp.bfloat16),
        in_specs=[
            pl.BlockSpec(memory_space=pltpu.SMEM),
            pl.BlockSpec(memory_space=pltpu.VMEM),
            pl.BlockSpec(memory_space=pltpu.VMEM),
        ],
        out_specs=pl.BlockSpec(memory_space=pltpu.VMEM),
        scratch_shapes=[
            pltpu.VMEM((N, D), jnp.bfloat16),
            pltpu.SemaphoreType.DMA((NCHUNK,)),
            pltpu.SemaphoreType.DMA((NCHUNK,)),
        ],
        compiler_params=pltpu.CompilerParams(collective_id=0),
    )(k_arr, keep_buf, send_buf)


# device time: 78492 ns/iter; 5.5071x vs baseline; 5.5071x over previous
import jax
import jax.numpy as jnp
from jax import lax
from jax.experimental import pallas as pl
from jax.experimental.pallas import tpu as pltpu

N = 4096
D = 1024
C = 128
NCHUNK = N // C
TAIL = NCHUNK
NSEM = NCHUNK + 1
UNROLL = 8


def _body(k_ref, osend_ref, okeep_ref, x_ref, out_ref, send_buf,
          send_sems, recv_sems):
    my_x = lax.axis_index("x")
    my_y = lax.axis_index("y")
    my_z = lax.axis_index("z")
    nbr = (my_x, 1 - my_y, my_z)

    k = k_ref[0]
    m = N - k
    base_local = jnp.where(my_y == 0, 0, k)
    base_recv = jnp.where(my_y == 0, m, 0)
    base_dst = jnp.where(my_y == 0, 0, m)

    barrier = pltpu.get_barrier_semaphore()
    pl.semaphore_signal(
        barrier, inc=1, device_id=nbr, device_id_type=pl.DeviceIdType.MESH
    )
    pl.semaphore_wait(barrier, 1)

    def gather_send(i, _):
        send_buf[pl.ds(i, 1)] = x_ref[pl.ds(osend_ref[i], 1)]
        return 0

    def send_rdma(src_off, dst_off, sem):
        return pltpu.make_async_remote_copy(
            src_ref=send_buf.at[pl.ds(src_off, C)],
            dst_ref=out_ref.at[pl.ds(dst_off, C)],
            send_sem=send_sems.at[sem],
            recv_sem=recv_sems.at[sem],
            device_id=nbr,
            device_id_type=pl.DeviceIdType.MESH,
        )

    for j in range(NCHUNK):

        @pl.when((j + 1) * C <= k)
        def _(j=j):
            lax.fori_loop(j * C, (j + 1) * C, gather_send, 0, unroll=UNROLL)
            send_rdma(j * C, base_dst + j * C, j).start()

    @pl.when(k % C != 0)
    def _():
        lax.fori_loop((k // C) * C, k, gather_send, 0)
        send_rdma(k - C, base_dst + k - C, TAIL).start()

    def gather_keep(i, _):
        out_ref[pl.ds(base_local + i, 1)] = x_ref[pl.ds(okeep_ref[i], 1)]
        return 0

    for j in range(NCHUNK):

        @pl.when((j + 1) * C <= m)
        def _(j=j):
            lax.fori_loop(j * C, (j + 1) * C, gather_keep, 0, unroll=UNROLL)

    @pl.when(m % C != 0)
    def _():
        lax.fori_loop((m // C) * C, m, gather_keep, 0)

    for j in range(NCHUNK):

        @pl.when((j + 1) * C <= k)
        def _(j=j):
            send_rdma(j * C, base_recv + j * C, j).wait_recv()

    @pl.when(k % C != 0)
    def _():
        send_rdma(k - C, base_recv + k - C, TAIL).wait_recv()

    for j in range(NCHUNK):

        @pl.when((j + 1) * C <= k)
        def _(j=j):
            send_rdma(j * C, base_dst + j * C, j).wait_send()

    @pl.when(k % C != 0)
    def _():
        send_rdma(k - C, base_dst + k - C, TAIL).wait_send()


def kernel(x, dest):
    my_y = lax.axis_index("y")
    send_mask = (dest != my_y).astype(jnp.int32)
    k = jnp.sum(send_mask)
    order_send = jnp.argsort(1 - send_mask, stable=True)
    order_keep = jnp.argsort(send_mask, stable=True)

    x3 = x.astype(jnp.bfloat16).reshape(N, 8, D // 8)
    k_arr = jnp.reshape(k, (1,)).astype(jnp.int32)

    out3 = pl.pallas_call(
        _body,
        out_shape=jax.ShapeDtypeStruct((N, 8, D // 8), jnp.bfloat16),
        in_specs=[
            pl.BlockSpec(memory_space=pltpu.SMEM),
            pl.BlockSpec(memory_space=pltpu.SMEM),
            pl.BlockSpec(memory_space=pltpu.SMEM),
            pl.BlockSpec(memory_space=pltpu.VMEM),
        ],
        out_specs=pl.BlockSpec(memory_space=pltpu.VMEM),
        scratch_shapes=[
            pltpu.VMEM((N, 8, D // 8), jnp.bfloat16),
            pltpu.SemaphoreType.DMA((NSEM,)),
            pltpu.SemaphoreType.DMA((NSEM,)),
        ],
        compiler_params=pltpu.CompilerParams(collective_id=0),
    )(k_arr, order_send.astype(jnp.int32), order_keep.astype(jnp.int32), x3)
    return out3.reshape(N, D)


# device time: 72725 ns/iter; 5.9438x vs baseline; 1.0793x over previous
import jax
import jax.numpy as jnp
from jax import lax
from jax.experimental import pallas as pl
from jax.experimental.pallas import tpu as pltpu

N = 4096
D = 1024
C = 128
NCHUNK = N // C
TAIL = NCHUNK
NSEM = NCHUNK + 1
UNROLL = 8


def _body(k_ref, osend_ref, okeep_ref, x_ref, out_ref, send_buf,
          send_sems, recv_sems):
    my_x = lax.axis_index("x")
    my_y = lax.axis_index("y")
    my_z = lax.axis_index("z")
    nbr = (my_x, 1 - my_y, my_z)

    k = k_ref[0]
    m = N - k
    base_local = jnp.where(my_y == 0, 0, k)
    base_recv = jnp.where(my_y == 0, m, 0)
    base_dst = jnp.where(my_y == 0, 0, m)

    barrier = pltpu.get_barrier_semaphore()
    pl.semaphore_signal(
        barrier, inc=1, device_id=nbr, device_id_type=pl.DeviceIdType.MESH
    )
    pl.semaphore_wait(barrier, 1)

    def gather_send(i, _):
        send_buf[pl.ds(i, 1)] = x_ref[pl.ds(osend_ref[i], 1)]
        return 0

    def send_rdma(src_off, dst_off, sem):
        return pltpu.make_async_remote_copy(
            src_ref=send_buf.at[pl.ds(src_off, C)],
            dst_ref=out_ref.at[pl.ds(dst_off, C)],
            send_sem=send_sems.at[sem],
            recv_sem=recv_sems.at[sem],
            device_id=nbr,
            device_id_type=pl.DeviceIdType.MESH,
        )

    for j in range(NCHUNK):

        @pl.when((j + 1) * C <= k)
        def _(j=j):
            lax.fori_loop(j * C, (j + 1) * C, gather_send, 0, unroll=UNROLL)
            send_rdma(j * C, base_dst + j * C, j).start()

    @pl.when(k % C != 0)
    def _():
        lax.fori_loop((k // C) * C, k, gather_send, 0)
        send_rdma(k - C, base_dst + k - C, TAIL).start()

    def gather_keep(i, _):
        out_ref[pl.ds(base_local + i, 1)] = x_ref[pl.ds(okeep_ref[i], 1)]
        return 0

    for j in range(NCHUNK):

        @pl.when((j + 1) * C <= m)
        def _(j=j):
            lax.fori_loop(j * C, (j + 1) * C, gather_keep, 0, unroll=UNROLL)

    @pl.when(m % C != 0)
    def _():
        lax.fori_loop((m // C) * C, m, gather_keep, 0)

    for j in range(NCHUNK):

        @pl.when((j + 1) * C <= k)
        def _(j=j):
            send_rdma(j * C, base_recv + j * C, j).wait_recv()

    @pl.when(k % C != 0)
    def _():
        send_rdma(k - C, base_recv + k - C, TAIL).wait_recv()

    for j in range(NCHUNK):

        @pl.when((j + 1) * C <= k)
        def _(j=j):
            send_rdma(j * C, base_dst + j * C, j).wait_send()

    @pl.when(k % C != 0)
    def _():
        send_rdma(k - C, base_dst + k - C, TAIL).wait_send()


def kernel(x, dest):
    my_y = lax.axis_index("y")
    send_mask = (dest != my_y).astype(jnp.int32)
    k = jnp.sum(send_mask)
    order_keep = jnp.argsort(send_mask, stable=True)
    order_send = jnp.roll(order_keep, k)

    x3 = x.astype(jnp.bfloat16).reshape(N, 8, D // 8)
    k_arr = jnp.reshape(k, (1,)).astype(jnp.int32)

    out3 = pl.pallas_call(
        _body,
        out_shape=jax.ShapeDtypeStruct((N, 8, D // 8), jnp.bfloat16),
        in_specs=[
            pl.BlockSpec(memory_space=pltpu.SMEM),
            pl.BlockSpec(memory_space=pltpu.SMEM),
            pl.BlockSpec(memory_space=pltpu.SMEM),
            pl.BlockSpec(memory_space=pltpu.VMEM),
        ],
        out_specs=pl.BlockSpec(memory_space=pltpu.VMEM),
        scratch_shapes=[
            pltpu.VMEM((N, 8, D // 8), jnp.bfloat16),
            pltpu.SemaphoreType.DMA((NSEM,)),
            pltpu.SemaphoreType.DMA((NSEM,)),
        ],
        compiler_params=pltpu.CompilerParams(collective_id=0),
    )(k_arr, order_send.astype(jnp.int32), order_keep.astype(jnp.int32), x3)
    return out3.reshape(N, D)


# device time: 72453 ns/iter; 5.9661x vs baseline; 1.0038x over previous
import jax
import jax.numpy as jnp
from jax import lax
from jax.experimental import pallas as pl
from jax.experimental.pallas import tpu as pltpu

N = 4096
D = 1024
C = 128
NCHUNK = N // C
TAIL = NCHUNK
NSEM = NCHUNK + 1
UNROLL = 8


def _body(k_ref, okeep_ref, x_ref, out_ref, send_buf,
          send_sems, recv_sems):
    my_x = lax.axis_index("x")
    my_y = lax.axis_index("y")
    my_z = lax.axis_index("z")
    nbr = (my_x, 1 - my_y, my_z)

    k = k_ref[0]
    m = N - k
    base_local = jnp.where(my_y == 0, 0, k)
    base_recv = jnp.where(my_y == 0, m, 0)
    base_dst = jnp.where(my_y == 0, 0, m)

    def gather_send(i, _):
        send_buf[pl.ds(i, 1)] = x_ref[pl.ds(okeep_ref[m + i], 1)]
        return 0

    @pl.when(C <= k)
    def _():
        lax.fori_loop(0, C, gather_send, 0, unroll=UNROLL)

    barrier = pltpu.get_barrier_semaphore()
    pl.semaphore_signal(
        barrier, inc=1, device_id=nbr, device_id_type=pl.DeviceIdType.MESH
    )
    pl.semaphore_wait(barrier, 1)

    def send_rdma(src_off, dst_off, sem):
        return pltpu.make_async_remote_copy(
            src_ref=send_buf.at[pl.ds(src_off, C)],
            dst_ref=out_ref.at[pl.ds(dst_off, C)],
            send_sem=send_sems.at[sem],
            recv_sem=recv_sems.at[sem],
            device_id=nbr,
            device_id_type=pl.DeviceIdType.MESH,
        )

    for j in range(NCHUNK):

        @pl.when((j + 1) * C <= k)
        def _(j=j):
            if j > 0:
                lax.fori_loop(j * C, (j + 1) * C, gather_send, 0, unroll=UNROLL)
            send_rdma(j * C, base_dst + j * C, j).start()

    @pl.when(k % C != 0)
    def _():
        lax.fori_loop((k // C) * C, k, gather_send, 0)
        send_rdma(k - C, base_dst + k - C, TAIL).start()

    def gather_keep(i, _):
        out_ref[pl.ds(base_local + i, 1)] = x_ref[pl.ds(okeep_ref[i], 1)]
        return 0

    for j in range(NCHUNK):

        @pl.when((j + 1) * C <= m)
        def _(j=j):
            lax.fori_loop(j * C, (j + 1) * C, gather_keep, 0, unroll=UNROLL)

    @pl.when(m % C != 0)
    def _():
        lax.fori_loop((m // C) * C, m, gather_keep, 0)

    for j in range(NCHUNK):

        @pl.when((j + 1) * C <= k)
        def _(j=j):
            send_rdma(j * C, base_recv + j * C, j).wait_recv()

    @pl.when(k % C != 0)
    def _():
        send_rdma(k - C, base_recv + k - C, TAIL).wait_recv()

    for j in range(NCHUNK):

        @pl.when((j + 1) * C <= k)
        def _(j=j):
            send_rdma(j * C, base_dst + j * C, j).wait_send()

    @pl.when(k % C != 0)
    def _():
        send_rdma(k - C, base_dst + k - C, TAIL).wait_send()


def kernel(x, dest):
    my_y = lax.axis_index("y")
    send_mask = (dest != my_y).astype(jnp.int32)
    k = jnp.sum(send_mask)
    order_keep = jnp.argsort(send_mask, stable=True)

    x3 = x.astype(jnp.bfloat16).reshape(N, 8, D // 8)
    k_arr = jnp.reshape(k, (1,)).astype(jnp.int32)

    out3 = pl.pallas_call(
        _body,
        out_shape=jax.ShapeDtypeStruct((N, 8, D // 8), jnp.bfloat16),
        in_specs=[
            pl.BlockSpec(memory_space=pltpu.SMEM),
            pl.BlockSpec(memory_space=pltpu.SMEM),
            pl.BlockSpec(memory_space=pltpu.VMEM),
        ],
        out_specs=pl.BlockSpec(memory_space=pltpu.VMEM),
        scratch_shapes=[
            pltpu.VMEM((N, 8, D // 8), jnp.bfloat16),
            pltpu.SemaphoreType.DMA((NSEM,)),
            pltpu.SemaphoreType.DMA((NSEM,)),
        ],
        compiler_params=pltpu.CompilerParams(collective_id=0),
    )(k_arr, order_keep.astype(jnp.int32), x3)
    return out3.reshape(N, D)
